# baseline (speedup 1.0000x reference)
import jax
import jax.numpy as jnp
from jax import lax
from jax.experimental import pallas as pl
from jax.experimental.pallas import tpu as pltpu

_R = 2
_EPS = 1e-5


def _fused_kernel(x_ref, t_ref, p_ref, q_ref, o_ref):
    C, rows, grp = x_ref.shape
    _, D, R, SUB, LANE = o_ref.shape
    S = rows * grp
    r3 = _R * _R * _R
    total = float(S * r3 * C)
    sr3 = float(S * r3)

    x = x_ref[...]
    sx = jnp.sum(x, axis=(1, 2)).reshape(C, 1)
    sxx = jnp.sum(x * x, axis=(1, 2)).reshape(C, 1)

    a_c = p_ref[:, 0:1]
    b_c = p_ref[:, 1:2]
    bias = p_ref[:, 2:3]

    s1 = jnp.sum(sx * a_c) + sr3 * jnp.sum(bias)
    mean = s1 / total
    d = bias - mean
    s2 = (jnp.sum(sxx * b_c)
          + 2.0 * jnp.sum(sx * a_c * d)
          + sr3 * jnp.sum(d * d))
    inv_std = lax.rsqrt(s2 / total + _EPS)

    xs = x * inv_std

    xcat = jnp.concatenate([xs[c] for c in range(C)], axis=1)
    z = jnp.dot(xcat, t_ref[...],
                preferred_element_type=jnp.float32)

    cst = (inv_std * (q_ref[0:1, :] - mean * q_ref[1:2, :])
           + q_ref[2:3, :])
    z = z + cst

    for i in range(R):
        for o in range(C):
            io = i * C + o
            tile = z[:, io * LANE:(io + 1) * LANE]
            o_ref[o, :, i, :, :] = tile.reshape(D, SUB, LANE)


def kernel(x, conv_t_w, conv_t_b, gn_w, gn_b, pw_w):
    N, C, D, H, W = x.shape
    r = _R
    r3 = r * r * r
    S = D * H * W
    f32 = jnp.float32
    grp = r * W
    rows = S // grp
    lane = 4 * r * W
    io_n = r * C

    xf = x.reshape(N, C, rows, grp).astype(f32)

    wt = conv_t_w.reshape(C, r3).astype(f32)
    bias = conv_t_b.astype(f32)
    gamma = gn_w.astype(f32)
    beta = gn_b.astype(f32)
    wpw = pw_w.reshape(C, C).T.astype(f32)

    jj, kk, hh, ww = jnp.meshgrid(jnp.arange(r), jnp.arange(r),
                                  jnp.arange(r), jnp.arange(W),
                                  indexing="ij")
    lane_of = (((hh * r + jj) * W + ww) * r + kk).ravel()
    src_of = (hh * W + ww).ravel()
    jk_of = (jj * r + kk).ravel()
    t_full = jnp.zeros((C, grp, r, C, lane), f32)
    for i in range(r):
        wg = wt[:, i * 4 + jk_of] * gamma[:, None]
        vals = wg[:, None, :] * wpw[:, :, None]
        t_full = t_full.at[:, src_of, i, :, lane_of].add(
            jnp.transpose(vals, (2, 0, 1)))
    t_big = t_full.reshape(C * grp, io_n * lane)

    a_vec = jnp.sum(wt, axis=1)
    b_vec = jnp.sum(wt * wt, axis=1)
    zero = jnp.zeros((C,), f32)
    p_cols = jnp.stack([a_vec, b_vec, bias] + [zero] * 5, axis=1)

    q1 = (bias * gamma) @ wpw
    q2 = gamma @ wpw
    q3 = beta @ wpw
    ones_l = jnp.ones((1, lane), f32)

    def spread_io(q):
        q2d = jnp.concatenate([q, q], axis=0).reshape(io_n, 1)
        return (q2d * ones_l).reshape(1, io_n * lane)

    q_rows = jnp.concatenate(
        [spread_io(q1), spread_io(q2), spread_io(q3)], axis=0)

    out = pl.pallas_call(
        _fused_kernel,
        out_shape=jax.ShapeDtypeStruct((N, C, D, r, H * r // 4, lane), f32),
        grid=(N,),
        in_specs=[
            pl.BlockSpec((None, C, rows, grp), lambda n: (n, 0, 0, 0)),
            pl.BlockSpec((C * grp, io_n * lane), lambda n: (0, 0)),
            pl.BlockSpec((C, 8), lambda n: (0, 0)),
            pl.BlockSpec((3, io_n * lane), lambda n: (0, 0)),
        ],
        out_specs=pl.BlockSpec((None, C, D, r, H * r // 4, lane),
                               lambda n: (n, 0, 0, 0, 0, 0)),
        compiler_params=pltpu.CompilerParams(
            dimension_semantics=("parallel",)),
    )(xf, t_big, p_cols, q_rows)

    return out.reshape(N, C, D * r, H * r, W * r).astype(x.dtype)


# speedup vs baseline: 1.1400x; 1.1400x over previous
import jax
import jax.numpy as jnp
from jax import lax
from jax.experimental import pallas as pl
from jax.experimental.pallas import tpu as pltpu

_R = 2
_EPS = 1e-5


def _fused_kernel(x_ref, t_ref, p_ref, q_ref, o_ref):
    rows, CL = x_ref.shape
    C, D, R, SUB, LANE = o_ref.shape
    S = rows * CL // C
    r3 = _R * _R * _R
    total = float(S * r3 * C)
    sr3 = float(S * r3)

    x = x_ref[...]
    a512 = p_ref[0:1, :]
    bias512 = p_ref[1:2, :]
    b512 = p_ref[2:3, :]
    sumb = p_ref[3, 0]
    sumb2 = p_ref[3, 1]

    sax = jnp.sum(x * a512)
    sabx = jnp.sum(x * (a512 * bias512))
    sbxx = jnp.sum((x * x) * b512)

    s1 = sax + sr3 * sumb
    mean = s1 / total
    s2 = (sbxx
          + 2.0 * (sabx - mean * sax)
          + sr3 * (sumb2 - 2.0 * mean * sumb + C * mean * mean))
    inv_std = lax.rsqrt(s2 / total + _EPS)

    xs = x * inv_std
    z = jnp.dot(xs, t_ref[...],
                preferred_element_type=jnp.float32)

    cst = (inv_std * (q_ref[0:1, :] - mean * q_ref[1:2, :])
           + q_ref[2:3, :])
    z = z + cst

    for i in range(R):
        for o in range(C):
            io = i * C + o
            tile = z[:, io * LANE:(io + 1) * LANE]
            o_ref[o, :, i, :, :] = tile.reshape(D, SUB, LANE)


def kernel(x, conv_t_w, conv_t_b, gn_w, gn_b, pw_w):
    N, C, D, H, W = x.shape
    r = _R
    r3 = r * r * r
    S = D * H * W
    f32 = jnp.float32
    grp = r * W
    rows = S // grp
    lane = 4 * r * W
    io_n = r * C

    xf = jnp.transpose(x.reshape(N, C, rows, grp).astype(f32),
                       (0, 2, 1, 3)).reshape(N, rows, C * grp)

    wt = conv_t_w.reshape(C, r3).astype(f32)
    bias = conv_t_b.astype(f32)
    gamma = gn_w.astype(f32)
    beta = gn_b.astype(f32)
    wpw = pw_w.reshape(C, C).T.astype(f32)

    jj, kk, hh, ww = jnp.meshgrid(jnp.arange(r), jnp.arange(r),
                                  jnp.arange(r), jnp.arange(W),
                                  indexing="ij")
    lane_of = (((hh * r + jj) * W + ww) * r + kk).ravel()
    src_of = (hh * W + ww).ravel()
    jk_of = (jj * r + kk).ravel()
    t_full = jnp.zeros((C, grp, r, C, lane), f32)
    for i in range(r):
        wg = wt[:, i * 4 + jk_of] * gamma[:, None]
        vals = wg[:, None, :] * wpw[:, :, None]
        t_full = t_full.at[:, src_of, i, :, lane_of].add(
            jnp.transpose(vals, (2, 0, 1)))
    t_big = t_full.reshape(C * grp, io_n * lane)

    a_vec = jnp.sum(wt, axis=1)
    b_vec = jnp.sum(wt * wt, axis=1)
    ones_g = jnp.ones((1, grp), f32)
    a512 = (a_vec[:, None] * ones_g).reshape(1, C * grp)
    b512 = (b_vec[:, None] * ones_g).reshape(1, C * grp)
    bias512 = (bias[:, None] * ones_g).reshape(1, C * grp)
    scal = jnp.zeros((1, C * grp), f32)
    scal = scal.at[0, 0].set(jnp.sum(bias))
    scal = scal.at[0, 1].set(jnp.sum(bias * bias))
    p_rows = jnp.concatenate([a512, bias512, b512, scal], axis=0)

    q1 = (bias * gamma) @ wpw
    q2 = gamma @ wpw
    q3 = beta @ wpw
    ones_l = jnp.ones((1, lane), f32)

    def spread_io(q):
        q2d = jnp.concatenate([q, q], axis=0).reshape(io_n, 1)
        return (q2d * ones_l).reshape(1, io_n * lane)

    q_rows = jnp.concatenate(
        [spread_io(q1), spread_io(q2), spread_io(q3)], axis=0)

    out = pl.pallas_call(
        _fused_kernel,
        out_shape=jax.ShapeDtypeStruct((N, C, D, r, H * r // 4, lane), f32),
        grid=(N,),
        in_specs=[
            pl.BlockSpec((None, rows, C * grp), lambda n: (n, 0, 0)),
            pl.BlockSpec((C * grp, io_n * lane), lambda n: (0, 0)),
            pl.BlockSpec((4, C * grp), lambda n: (0, 0)),
            pl.BlockSpec((3, io_n * lane), lambda n: (0, 0)),
        ],
        out_specs=pl.BlockSpec((None, C, D, r, H * r // 4, lane),
                               lambda n: (n, 0, 0, 0, 0, 0)),
        compiler_params=pltpu.CompilerParams(
            dimension_semantics=("parallel",)),
    )(xf, t_big, p_rows, q_rows)

    return out.reshape(N, C, D * r, H * r, W * r).astype(x.dtype)


# speedup vs baseline: 1.1571x; 1.0150x over previous
import jax
import jax.numpy as jnp
from jax import lax
from jax.experimental import pallas as pl
from jax.experimental.pallas import tpu as pltpu

_R = 2
_EPS = 1e-5


def _fused_kernel(x_ref, t_ref, p_ref, q_ref, o_ref, xsc_ref):
    C, S = x_ref.shape
    _, rows, grp = xsc_ref.shape
    _, D, R, SUB, LANE = o_ref.shape
    r3 = _R * _R * _R
    total = float(S * r3 * C)
    sr3 = float(S * r3)

    x = x_ref[...]
    sx = jnp.sum(x, axis=1, keepdims=True)
    sxx = jnp.sum(x * x, axis=1, keepdims=True)

    a_c = p_ref[:, 0:1]
    b_c = p_ref[:, 1:2]
    bias = p_ref[:, 2:3]

    s1 = jnp.sum(sx * a_c) + sr3 * jnp.sum(bias)
    mean = s1 / total
    d = bias - mean
    s2 = (jnp.sum(sxx * b_c)
          + 2.0 * jnp.sum(sx * a_c * d)
          + sr3 * jnp.sum(d * d))
    inv_std = lax.rsqrt(s2 / total + _EPS)

    xsc_ref[...] = (x * inv_std).reshape(C, rows, grp)
    xs = jnp.concatenate([xsc_ref[c] for c in range(C)],
                         axis=1)
    z = jnp.dot(xs, t_ref[...],
                preferred_element_type=jnp.float32)

    cst = (inv_std * (q_ref[0:1, :] - mean * q_ref[1:2, :])
           + q_ref[2:3, :])
    z = z + cst

    for i in range(R):
        for o in range(C):
            io = i * C + o
            tile = z[:, io * LANE:(io + 1) * LANE]
            o_ref[o, :, i, :, :] = tile.reshape(D, SUB, LANE)


def kernel(x, conv_t_w, conv_t_b, gn_w, gn_b, pw_w):
    N, C, D, H, W = x.shape
    r = _R
    r3 = r * r * r
    S = D * H * W
    f32 = jnp.float32
    grp = r * W
    rows = S // grp
    lane = 4 * r * W
    io_n = r * C

    xf = x.reshape(N, C, S).astype(f32)

    wt = conv_t_w.reshape(C, r3).astype(f32)
    bias = conv_t_b.astype(f32)
    gamma = gn_w.astype(f32)
    beta = gn_b.astype(f32)
    wpw = pw_w.reshape(C, C).T.astype(f32)

    jj, kk, hh, ww = jnp.meshgrid(jnp.arange(r), jnp.arange(r),
                                  jnp.arange(r), jnp.arange(W),
                                  indexing="ij")
    lane_of = (((hh * r + jj) * W + ww) * r + kk).ravel()
    src_of = (hh * W + ww).ravel()
    jk_of = (jj * r + kk).ravel()
    t_full = jnp.zeros((C, grp, r, C, lane), f32)
    for i in range(r):
        wg = wt[:, i * 4 + jk_of] * gamma[:, None]
        vals = wg[:, None, :] * wpw[:, :, None]
        t_full = t_full.at[:, src_of, i, :, lane_of].add(
            jnp.transpose(vals, (2, 0, 1)))
    t_big = t_full.reshape(C * grp, io_n * lane)

    a_vec = jnp.sum(wt, axis=1)
    b_vec = jnp.sum(wt * wt, axis=1)
    zero = jnp.zeros((C,), f32)
    p_rows = jnp.stack([a_vec, b_vec, bias] + [zero] * 5, axis=1)

    q1 = (bias * gamma) @ wpw
    q2 = gamma @ wpw
    q3 = beta @ wpw
    ones_l = jnp.ones((1, lane), f32)

    def spread_io(q):
        q2d = jnp.concatenate([q, q], axis=0).reshape(io_n, 1)
        return (q2d * ones_l).reshape(1, io_n * lane)

    q_rows = jnp.concatenate(
        [spread_io(q1), spread_io(q2), spread_io(q3)], axis=0)

    out = pl.pallas_call(
        _fused_kernel,
        out_shape=jax.ShapeDtypeStruct((N, C, D, r, H * r // 4, lane), f32),
        grid=(N,),
        in_specs=[
            pl.BlockSpec((None, C, S), lambda n: (n, 0, 0)),
            pl.BlockSpec((C * grp, io_n * lane), lambda n: (0, 0)),
            pl.BlockSpec((C, 8), lambda n: (0, 0)),
            pl.BlockSpec((3, io_n * lane), lambda n: (0, 0)),
        ],
        out_specs=pl.BlockSpec((None, C, D, r, H * r // 4, lane),
                               lambda n: (n, 0, 0, 0, 0, 0)),
        scratch_shapes=[pltpu.VMEM((C, rows, grp), f32)],
        compiler_params=pltpu.CompilerParams(
            dimension_semantics=("parallel",)),
    )(xf, t_big, p_rows, q_rows)

    return out.reshape(N, C, D * r, H * r, W * r).astype(x.dtype)
